# baseline (speedup 1.0000x reference)
import functools

import jax
import jax.numpy as jnp
from jax import lax
from jax.experimental import pallas as pl
from jax.experimental.pallas import tpu as pltpu
from jax.experimental.pallas import tpu_sc as plsc

_EPS = 1e-5


def _k1_body(f_ref, ws_ref, wi_ref, wr_ref, bs_ref, bi_ref, br_ref,
             zs_ref, zi_ref, zr_ref, stats_ref):
    @pl.when(pl.program_id(0) == 0)
    def _():
        stats_ref[...] = jnp.zeros_like(stats_ref)

    f = f_ref[...]
    dn = (((1,), (1,)), ((), ()))
    zs = lax.dot_general(f, ws_ref[...], dn, preferred_element_type=jnp.float32) + f + bs_ref[...]
    zi = lax.dot_general(f, wi_ref[...], dn, preferred_element_type=jnp.float32) + f + bi_ref[...]
    zr = lax.dot_general(f, wr_ref[...], dn, preferred_element_type=jnp.float32) + f + br_ref[...]
    zs_ref[...] = zs
    zi_ref[...] = zi
    zr_ref[...] = zr
    upd = jnp.concatenate(
        [jnp.sum(zs, 0, keepdims=True), jnp.sum(zs * zs, 0, keepdims=True),
         jnp.sum(zi, 0, keepdims=True), jnp.sum(zi * zi, 0, keepdims=True),
         jnp.sum(zr, 0, keepdims=True), jnp.sum(zr * zr, 0, keepdims=True),
         jnp.zeros((2, zs.shape[1]), jnp.float32)], axis=0)
    stats_ref[...] += upd


def _bn_coefs(stats, row, gamma, beta, n):
    mean = stats[row:row + 1, :] * (1.0 / n)
    var = stats[row + 1:row + 2, :] * (1.0 / n) - mean * mean
    sc = gamma * lax.rsqrt(var + _EPS)
    sh = beta - mean * sc
    return sc, sh


def _k2_body(n, zi_ref, stats_ref, g_ref, b_ref, i_ref):
    sc, sh = _bn_coefs(stats_ref[...], 2, g_ref[...], b_ref[...], n)
    i_ref[...] = jnp.maximum(zi_ref[...] * sc + sh, 0.0)


def _k3_body(n, zs_ref, zr_ref, i_ref, n0_ref, n1_ref, stats_ref, g_ref, b_ref,
             a_ref, bm_ref, rw_ref, o1_ref, o2_ref, o3_ref,
             tib_ref, trb_ref, ob_ref, out_ref):
    st = stats_ref[...]
    g = g_ref[...]
    be = b_ref[...]
    sc_s, sh_s = _bn_coefs(st, 0, g, be, n)
    sc_r, sh_r = _bn_coefs(st, 4, g, be, n)
    s = jnp.maximum(zs_ref[...] * sc_s + sh_s, 0.0)
    r = jnp.maximum(zr_ref[...] * sc_r + sh_r, 0.0)
    i = i_ref[...]
    nb = n0_ref[...] + n1_ref[...]

    o1 = o1_ref[...]
    o2 = o2_ref[...]
    o3 = o3_ref[...]
    p21 = o2 - o1
    p32 = o3 - o2
    dnm = (((1,), (0,)), ((), ()))
    dnt = (((1,), (1,)), ((), ()))
    g_s = o1 + lax.dot_general(p21, a_ref[...], dnm, preferred_element_type=jnp.float32)
    g_i = o2 + lax.dot_general(p32, rw_ref[...], dnm, preferred_element_type=jnp.float32)
    g_n = lax.dot_general(p21, bm_ref[...], dnm, preferred_element_type=jnp.float32)
    c = (lax.dot_general(tib_ref[...], p21, dnt, preferred_element_type=jnp.float32)
         + lax.dot_general(trb_ref[...], p32, dnt, preferred_element_type=jnp.float32)
         + ob_ref[...])

    x = (lax.dot_general(s, g_s, dnt, preferred_element_type=jnp.float32)
         + lax.dot_general(i, g_i, dnt, preferred_element_type=jnp.float32)
         + lax.dot_general(r, o3, dnt, preferred_element_type=jnp.float32)
         + lax.dot_general(nb, g_n, dnt, preferred_element_type=jnp.float32)
         + c)
    x = x - jnp.max(x, axis=-1, keepdims=True)
    e = jnp.exp(x)
    out_ref[...] = e / jnp.sum(e, axis=-1, keepdims=True)


@functools.cache
def _make_sc_scatter(n, d, e, nc, ns, c):
    nw = nc * ns
    epw = e // nw
    nchunk = epw // c
    npc = 25
    nstage = nchunk // npc
    rpt = (n // ns) // 8 * 8
    tail = n - rpt * ns
    mesh = plsc.VectorSubcoreMesh(core_axis_name="c", subcore_axis_name="s")

    @functools.partial(
        pl.kernel,
        out_type=jax.ShapeDtypeStruct((nc, n, d), jnp.float32),
        mesh=mesh,
        scratch_types=[
            pltpu.VMEM((npc, c), jnp.int32),
            pltpu.VMEM((npc, c), jnp.int32),
            pltpu.VMEM((npc, c), jnp.float32),
            pltpu.VMEM((c, d), jnp.float32),
            pltpu.VMEM((c, d), jnp.float32),
            pltpu.VMEM((c, d), jnp.float32),
            pltpu.VMEM_SHARED((n, d), jnp.float32),
            pltpu.SemaphoreType.DMA,
            pltpu.SemaphoreType.DMA,
            pltpu.SemaphoreType.DMA,
            pltpu.SemaphoreType.DMA,
            pltpu.SemaphoreType.DMA,
            pltpu.SemaphoreType.DMA,
        ],
    )
    def sc_scatter(i_hbm, src_hbm, dst_hbm, w_hbm, z_hbm, out_hbm,
                   src_v, dst_v, w_v, rows0_v, rows1_v, rows2_v, acc,
                   gs0, gs1, gs2, ss0, ss1, ss2):
        ci = lax.axis_index("c")
        si = lax.axis_index("s")
        wid = si * nc + ci
        r0 = pl.multiple_of(si * rpt, 8)
        pltpu.sync_copy(z_hbm.at[pl.ds(r0, rpt)], acc.at[pl.ds(r0, rpt)])
        if tail:
            @pl.when(si == ns - 1)
            def _():
                pltpu.sync_copy(z_hbm.at[pl.ds(rpt * ns, tail)],
                                acc.at[pl.ds(rpt * ns, tail)])
        plsc.subcore_barrier()

        rows = (rows0_v, rows1_v, rows2_v)
        gsem = (gs0, gs1, gs2)
        ssem = (ss0, ss1, ss2)

        def stage(st, carry):
            pltpu.sync_copy(src_hbm.at[wid, st], src_v)
            pltpu.sync_copy(dst_hbm.at[wid, st], dst_v)
            pltpu.sync_copy(w_hbm.at[wid, st], w_v)
            pltpu.async_copy(i_hbm.at[src_v.at[0]], rows0_v, gs0)

            def body(j, bi):
                cur = rows[bi]
                ni = (bi + 1) % 3
                pltpu.make_async_copy(i_hbm.at[src_v.at[j]], cur, gsem[bi]).wait()

                @pl.when(j >= 2)
                def _():
                    pltpu.make_async_copy(
                        rows[ni], acc.at[dst_v.at[j]], ssem[ni]).wait()

                @pl.when(j < npc - 1)
                def _():
                    pltpu.async_copy(i_hbm.at[src_v.at[j + 1]], rows[ni], gsem[ni])

                for g in range(c // 16):
                    wv = w_v[j, pl.ds(g * 16, 16)]
                    for e in range(16):
                        w = wv[e]
                        ei = g * 16 + e
                        for k in range(d // 16):
                            sl = pl.ds(k * 16, 16)
                            cur[ei, sl] = cur[ei, sl] * w
                pltpu.async_copy(cur, acc.at[dst_v.at[j]], ssem[bi], add=True)

            def chunk(j, c2):
                for b in range(3):
                    @pl.when(j % 3 == b)
                    def _(b=b):
                        body(j, b)

                return c2

            lax.fori_loop(0, npc, chunk, 0)
            for jj in (npc - 2, npc - 1):
                pltpu.make_async_copy(
                    rows[jj % 3], acc.at[dst_v.at[jj]], ssem[jj % 3]).wait()
            return carry

        lax.fori_loop(0, nstage, stage, 0)
        plsc.subcore_barrier()
        pltpu.sync_copy(acc.at[pl.ds(r0, rpt)], out_hbm.at[ci, pl.ds(r0, rpt)])
        if tail:
            @pl.when(si == ns - 1)
            def _():
                pltpu.sync_copy(acc.at[pl.ds(rpt * ns, tail)],
                                out_hbm.at[ci, pl.ds(rpt * ns, tail)])

    return sc_scatter


def _sc_partials(i_arr, src3, dst3, w3, zeros, nc, ns, c):
    n, d = i_arr.shape
    e = src3.size
    return _make_sc_scatter(n, d, e, nc, ns, c)(i_arr, src3, dst3, w3, zeros)


def kernel(feature, edge_index, edge_weight, W_s, b_s, W_i, b_i, W_r, b_r,
           bn_gamma, bn_beta, toI_W, toI_b, toR_W, toR_b, out_W, out_b):
    n, d = feature.shape
    e = edge_weight.shape[0]
    blk = 1000 if n % 1000 == 0 else n
    nblk = n // blk
    nf = float(n)

    bs = b_s.reshape(1, d)
    bi = b_i.reshape(1, d)
    br = b_r.reshape(1, d)
    gam = bn_gamma.reshape(1, d)
    bet = bn_beta.reshape(1, d)

    row = lambda i: pl.BlockSpec((blk, d), lambda b: (b, 0))
    full = lambda s: pl.BlockSpec(s, lambda b: (0,) * len(s))

    z_s, z_i, z_r, stats = pl.pallas_call(
        _k1_body,
        grid=(nblk,),
        in_specs=[row(0)] + [full((d, d))] * 3 + [full((1, d))] * 3,
        out_specs=[row(0), row(0), row(0), full((8, d))],
        out_shape=[jax.ShapeDtypeStruct((n, d), jnp.float32)] * 3
        + [jax.ShapeDtypeStruct((8, d), jnp.float32)],
    )(feature, W_s, W_i, W_r, bs, bi, br)

    i_arr = pl.pallas_call(
        functools.partial(_k2_body, nf),
        grid=(nblk,),
        in_specs=[row(0), full((8, d)), full((1, d)), full((1, d))],
        out_specs=row(0),
        out_shape=jax.ShapeDtypeStruct((n, d), jnp.float32),
    )(z_i, stats, gam, bet)

    info = plsc.get_sparse_core_info()
    nc, ns = info.num_cores, info.num_subcores
    c = 80
    npc = 25
    src3 = edge_index[0].reshape(nc * ns, -1, npc, c)
    dst3 = edge_index[1].reshape(nc * ns, -1, npc, c)
    w3 = edge_weight.reshape(nc * ns, -1, npc, c)
    zeros = jnp.zeros((n, d), jnp.float32)
    partials = _sc_partials(i_arr, src3, dst3, w3, zeros, nc, ns, c)

    a_m = toI_W[:, :d]
    b_m = toI_W[:, d:]
    o1 = out_W[:, :d]
    o2 = out_W[:, d:2 * d]
    o3 = out_W[:, 2 * d:]

    out = pl.pallas_call(
        functools.partial(_k3_body, nf),
        grid=(nblk,),
        in_specs=[row(0)] * 5 + [full((8, d)), full((1, d)), full((1, d))]
        + [full((d, d))] * 3 + [full((3, d))] * 3
        + [full((1, d)), full((1, d)), full((1, 3))],
        out_specs=pl.BlockSpec((blk, 3), lambda b: (b, 0)),
        out_shape=jax.ShapeDtypeStruct((n, 3), jnp.float32),
    )(z_s, z_r, i_arr, partials[0], partials[1], stats, gam, bet,
      a_m, b_m, toR_W, o1, o2, o3,
      toI_b.reshape(1, d), toR_b.reshape(1, d), out_b.reshape(1, 3))
    return out


# speedup vs baseline: 1.1040x; 1.1040x over previous
import functools

import jax
import jax.numpy as jnp
from jax import lax
from jax.experimental import pallas as pl
from jax.experimental.pallas import tpu as pltpu
from jax.experimental.pallas import tpu_sc as plsc

_EPS = 1e-5


def _k1_body(f_ref, ws_ref, wi_ref, wr_ref, bs_ref, bi_ref, br_ref,
             zs_ref, zi_ref, zr_ref, stats_ref):
    @pl.when(pl.program_id(0) == 0)
    def _():
        stats_ref[...] = jnp.zeros_like(stats_ref)

    f = f_ref[...]
    dn = (((1,), (1,)), ((), ()))
    zs = lax.dot_general(f, ws_ref[...], dn, preferred_element_type=jnp.float32) + f + bs_ref[...]
    zi = lax.dot_general(f, wi_ref[...], dn, preferred_element_type=jnp.float32) + f + bi_ref[...]
    zr = lax.dot_general(f, wr_ref[...], dn, preferred_element_type=jnp.float32) + f + br_ref[...]
    zs_ref[...] = zs
    zi_ref[...] = zi
    zr_ref[...] = zr
    upd = jnp.concatenate(
        [jnp.sum(zs, 0, keepdims=True), jnp.sum(zs * zs, 0, keepdims=True),
         jnp.sum(zi, 0, keepdims=True), jnp.sum(zi * zi, 0, keepdims=True),
         jnp.sum(zr, 0, keepdims=True), jnp.sum(zr * zr, 0, keepdims=True),
         jnp.zeros((2, zs.shape[1]), jnp.float32)], axis=0)
    stats_ref[...] += upd


def _bn_coefs(stats, row, gamma, beta, n):
    mean = stats[row:row + 1, :] * (1.0 / n)
    var = stats[row + 1:row + 2, :] * (1.0 / n) - mean * mean
    sc = gamma * lax.rsqrt(var + _EPS)
    sh = beta - mean * sc
    return sc, sh


def _k2_body(n, zi_ref, stats_ref, g_ref, b_ref, i_ref):
    sc, sh = _bn_coefs(stats_ref[...], 2, g_ref[...], b_ref[...], n)
    i_ref[...] = jnp.maximum(zi_ref[...] * sc + sh, 0.0)


def _k3_body(n, zs_ref, zr_ref, i_ref, n0_ref, n1_ref, stats_ref, g_ref, b_ref,
             a_ref, bm_ref, rw_ref, o1_ref, o2_ref, o3_ref,
             tib_ref, trb_ref, ob_ref, out_ref):
    st = stats_ref[...]
    g = g_ref[...]
    be = b_ref[...]
    sc_s, sh_s = _bn_coefs(st, 0, g, be, n)
    sc_r, sh_r = _bn_coefs(st, 4, g, be, n)
    s = jnp.maximum(zs_ref[...] * sc_s + sh_s, 0.0)
    r = jnp.maximum(zr_ref[...] * sc_r + sh_r, 0.0)
    i = i_ref[...]
    nb = n0_ref[...] + n1_ref[...]

    o1 = o1_ref[...]
    o2 = o2_ref[...]
    o3 = o3_ref[...]
    p21 = o2 - o1
    p32 = o3 - o2
    dnm = (((1,), (0,)), ((), ()))
    dnt = (((1,), (1,)), ((), ()))
    g_s = o1 + lax.dot_general(p21, a_ref[...], dnm, preferred_element_type=jnp.float32)
    g_i = o2 + lax.dot_general(p32, rw_ref[...], dnm, preferred_element_type=jnp.float32)
    g_n = lax.dot_general(p21, bm_ref[...], dnm, preferred_element_type=jnp.float32)
    c = (lax.dot_general(tib_ref[...], p21, dnt, preferred_element_type=jnp.float32)
         + lax.dot_general(trb_ref[...], p32, dnt, preferred_element_type=jnp.float32)
         + ob_ref[...])

    x = (lax.dot_general(s, g_s, dnt, preferred_element_type=jnp.float32)
         + lax.dot_general(i, g_i, dnt, preferred_element_type=jnp.float32)
         + lax.dot_general(r, o3, dnt, preferred_element_type=jnp.float32)
         + lax.dot_general(nb, g_n, dnt, preferred_element_type=jnp.float32)
         + c)
    x = x - jnp.max(x, axis=-1, keepdims=True)
    e = jnp.exp(x)
    out_ref[...] = e / jnp.sum(e, axis=-1, keepdims=True)


@functools.cache
def _make_sc_scatter(n, d, e, nc, ns, c):
    nw = nc * ns
    epw = e // nw
    nchunk = epw // c
    npc = 25
    nstage = nchunk // npc
    rpt = (n // ns) // 8 * 8
    tail = n - rpt * ns
    mesh = plsc.VectorSubcoreMesh(core_axis_name="c", subcore_axis_name="s")

    @functools.partial(
        pl.kernel,
        out_type=jax.ShapeDtypeStruct((nc, n, d), jnp.float32),
        mesh=mesh,
        scratch_types=[
            pltpu.VMEM((npc, c), jnp.int32),
            pltpu.VMEM((npc, 48), jnp.int32),
            pltpu.VMEM((npc, 32), jnp.int32),
            pltpu.VMEM((npc, c), jnp.float32),
            pltpu.VMEM((c, d), jnp.float32),
            pltpu.VMEM((c, d), jnp.float32),
            pltpu.VMEM_SHARED((n, d), jnp.float32),
            pltpu.SemaphoreType.DMA,
            pltpu.SemaphoreType.DMA,
            pltpu.SemaphoreType.DMA,
            pltpu.SemaphoreType.DMA,
            pltpu.SemaphoreType.DMA,
            pltpu.SemaphoreType.DMA,
        ],
    )
    def sc_scatter(i_hbm, src_hbm, dsta_hbm, dstb_hbm, w_hbm, z_hbm, out_hbm,
                   src_v, dsta_v, dstb_v, w_v, rows0_v, rows1_v, acc,
                   gs0, gs1, sa0, sb0, sa1, sb1):
        ci = lax.axis_index("c")
        si = lax.axis_index("s")
        wid = si * nc + ci
        r0 = pl.multiple_of(si * rpt, 8)
        pltpu.sync_copy(z_hbm.at[pl.ds(r0, rpt)], acc.at[pl.ds(r0, rpt)])
        if tail:
            @pl.when(si == ns - 1)
            def _():
                pltpu.sync_copy(z_hbm.at[pl.ds(rpt * ns, tail)],
                                acc.at[pl.ds(rpt * ns, tail)])
        plsc.subcore_barrier()

        rows = (rows0_v, rows1_v)
        gsem = (gs0, gs1)
        ssem = ((sa0, sb0), (sa1, sb1))
        halves = ((dsta_v, 0, 48), (dstb_v, 48, 32))

        def scat_start(buf, j, half, sem):
            idx, off, hn = halves[half]
            pltpu.async_copy(
                buf.at[pl.ds(off, hn)], acc.at[idx.at[j]], sem, add=True)

        def scat_wait(buf, j, half, sem):
            idx, off, hn = halves[half]
            pltpu.make_async_copy(
                buf.at[pl.ds(off, hn)], acc.at[idx.at[j]], sem).wait()

        def stage(st, carry):
            pltpu.sync_copy(src_hbm.at[wid, st], src_v)
            pltpu.sync_copy(dsta_hbm.at[wid, st], dsta_v)
            pltpu.sync_copy(dstb_hbm.at[wid, st], dstb_v)
            pltpu.sync_copy(w_hbm.at[wid, st], w_v)
            pltpu.async_copy(i_hbm.at[src_v.at[0]], rows0_v, gs0)

            def scale_edges(cur, j, e0, e1):
                for g in range(e0 // 16, e1 // 16):
                    wv = w_v[j, pl.ds(g * 16, 16)]
                    for e in range(16):
                        w = wv[e]
                        ei = g * 16 + e
                        for k in range(d // 16):
                            sl = pl.ds(k * 16, 16)
                            cur[ei, sl] = cur[ei, sl] * w

            def body(j, bi):
                cur = rows[bi]
                ni = 1 - bi
                pltpu.make_async_copy(i_hbm.at[src_v.at[j]], cur, gsem[bi]).wait()

                @pl.when(j >= 1)
                def _():
                    scat_wait(rows[ni], j, 0, ssem[ni][0])
                    scat_wait(rows[ni], j, 1, ssem[ni][1])

                @pl.when(j < npc - 1)
                def _():
                    pltpu.async_copy(i_hbm.at[src_v.at[j + 1]], rows[ni], gsem[ni])

                scale_edges(cur, j, 0, 48)
                scat_start(cur, j, 0, ssem[bi][0])
                scale_edges(cur, j, 48, 80)
                scat_start(cur, j, 1, ssem[bi][1])

            def chunk(j, c2):
                for b in range(2):
                    @pl.when(j % 2 == b)
                    def _(b=b):
                        body(j, b)

                return c2

            lax.fori_loop(0, npc, chunk, 0)
            bl = (npc - 1) % 2
            scat_wait(rows[bl], npc - 1, 0, ssem[bl][0])
            scat_wait(rows[bl], npc - 1, 1, ssem[bl][1])
            return carry

        lax.fori_loop(0, nstage, stage, 0)
        plsc.subcore_barrier()
        pltpu.sync_copy(acc.at[pl.ds(r0, rpt)], out_hbm.at[ci, pl.ds(r0, rpt)])
        if tail:
            @pl.when(si == ns - 1)
            def _():
                pltpu.sync_copy(acc.at[pl.ds(rpt * ns, tail)],
                                out_hbm.at[ci, pl.ds(rpt * ns, tail)])

    return sc_scatter


def _sc_partials(i_arr, src3, dsta, dstb, w3, zeros, nc, ns, c):
    n, d = i_arr.shape
    e = src3.size
    return _make_sc_scatter(n, d, e, nc, ns, c)(i_arr, src3, dsta, dstb, w3, zeros)


def kernel(feature, edge_index, edge_weight, W_s, b_s, W_i, b_i, W_r, b_r,
           bn_gamma, bn_beta, toI_W, toI_b, toR_W, toR_b, out_W, out_b):
    n, d = feature.shape
    e = edge_weight.shape[0]
    blk = 1000 if n % 1000 == 0 else n
    nblk = n // blk
    nf = float(n)

    bs = b_s.reshape(1, d)
    bi = b_i.reshape(1, d)
    br = b_r.reshape(1, d)
    gam = bn_gamma.reshape(1, d)
    bet = bn_beta.reshape(1, d)

    row = lambda i: pl.BlockSpec((blk, d), lambda b: (b, 0))
    full = lambda s: pl.BlockSpec(s, lambda b: (0,) * len(s))

    z_s, z_i, z_r, stats = pl.pallas_call(
        _k1_body,
        grid=(nblk,),
        in_specs=[row(0)] + [full((d, d))] * 3 + [full((1, d))] * 3,
        out_specs=[row(0), row(0), row(0), full((8, d))],
        out_shape=[jax.ShapeDtypeStruct((n, d), jnp.float32)] * 3
        + [jax.ShapeDtypeStruct((8, d), jnp.float32)],
    )(feature, W_s, W_i, W_r, bs, bi, br)

    i_arr = pl.pallas_call(
        functools.partial(_k2_body, nf),
        grid=(nblk,),
        in_specs=[row(0), full((8, d)), full((1, d)), full((1, d))],
        out_specs=row(0),
        out_shape=jax.ShapeDtypeStruct((n, d), jnp.float32),
    )(z_i, stats, gam, bet)

    info = plsc.get_sparse_core_info()
    nc, ns = info.num_cores, info.num_subcores
    c = 80
    npc = 25
    src3 = edge_index[0].reshape(nc * ns, -1, npc, c)
    dst4 = edge_index[1].reshape(nc * ns, -1, npc, c)
    dsta = dst4[..., :48]
    dstb = dst4[..., 48:]
    w3 = edge_weight.reshape(nc * ns, -1, npc, c)
    zeros = jnp.zeros((n, d), jnp.float32)
    partials = _sc_partials(i_arr, src3, dsta, dstb, w3, zeros, nc, ns, c)

    a_m = toI_W[:, :d]
    b_m = toI_W[:, d:]
    o1 = out_W[:, :d]
    o2 = out_W[:, d:2 * d]
    o3 = out_W[:, 2 * d:]

    out = pl.pallas_call(
        functools.partial(_k3_body, nf),
        grid=(nblk,),
        in_specs=[row(0)] * 5 + [full((8, d)), full((1, d)), full((1, d))]
        + [full((d, d))] * 3 + [full((3, d))] * 3
        + [full((1, d)), full((1, d)), full((1, 3))],
        out_specs=pl.BlockSpec((blk, 3), lambda b: (b, 0)),
        out_shape=jax.ShapeDtypeStruct((n, 3), jnp.float32),
    )(z_s, z_r, i_arr, partials[0], partials[1], stats, gam, bet,
      a_m, b_m, toR_W, o1, o2, o3,
      toI_b.reshape(1, d), toR_b.reshape(1, d), out_b.reshape(1, 3))
    return out


# speedup vs baseline: 1.1724x; 1.0620x over previous
import functools

import jax
import jax.numpy as jnp
from jax import lax
from jax.experimental import pallas as pl
from jax.experimental.pallas import tpu as pltpu
from jax.experimental.pallas import tpu_sc as plsc

_EPS = 1e-5


def _k1_body(f_ref, ws_ref, wi_ref, wr_ref, bs_ref, bi_ref, br_ref,
             zs_ref, zi_ref, zr_ref, stats_ref):
    @pl.when(pl.program_id(0) == 0)
    def _():
        stats_ref[...] = jnp.zeros_like(stats_ref)

    f = f_ref[...]
    dn = (((1,), (1,)), ((), ()))
    zs = lax.dot_general(f, ws_ref[...], dn, preferred_element_type=jnp.float32) + f + bs_ref[...]
    zi = lax.dot_general(f, wi_ref[...], dn, preferred_element_type=jnp.float32) + f + bi_ref[...]
    zr = lax.dot_general(f, wr_ref[...], dn, preferred_element_type=jnp.float32) + f + br_ref[...]
    zs_ref[...] = zs
    zi_ref[...] = zi
    zr_ref[...] = zr
    upd = jnp.concatenate(
        [jnp.sum(zs, 0, keepdims=True), jnp.sum(zs * zs, 0, keepdims=True),
         jnp.sum(zi, 0, keepdims=True), jnp.sum(zi * zi, 0, keepdims=True),
         jnp.sum(zr, 0, keepdims=True), jnp.sum(zr * zr, 0, keepdims=True),
         jnp.zeros((2, zs.shape[1]), jnp.float32)], axis=0)
    stats_ref[...] += upd


def _bn_coefs(stats, row, gamma, beta, n):
    mean = stats[row:row + 1, :] * (1.0 / n)
    var = stats[row + 1:row + 2, :] * (1.0 / n) - mean * mean
    sc = gamma * lax.rsqrt(var + _EPS)
    sh = beta - mean * sc
    return sc, sh


def _k3_body(n, zs_ref, zr_ref, zi_ref, n0_ref, n1_ref, stats_ref, g_ref, b_ref,
             a_ref, bm_ref, rw_ref, o1_ref, o2_ref, o3_ref,
             tib_ref, trb_ref, ob_ref, out_ref):
    st = stats_ref[...]
    g = g_ref[...]
    be = b_ref[...]
    sc_s, sh_s = _bn_coefs(st, 0, g, be, n)
    sc_i, sh_i = _bn_coefs(st, 2, g, be, n)
    sc_r, sh_r = _bn_coefs(st, 4, g, be, n)
    s = jnp.maximum(zs_ref[...] * sc_s + sh_s, 0.0)
    r = jnp.maximum(zr_ref[...] * sc_r + sh_r, 0.0)
    i = jnp.maximum(zi_ref[...] * sc_i + sh_i, 0.0)
    nb = n0_ref[...] + n1_ref[...]

    o1 = o1_ref[...]
    o2 = o2_ref[...]
    o3 = o3_ref[...]
    p21 = o2 - o1
    p32 = o3 - o2
    dnm = (((1,), (0,)), ((), ()))
    dnt = (((1,), (1,)), ((), ()))
    g_s = o1 + lax.dot_general(p21, a_ref[...], dnm, preferred_element_type=jnp.float32)
    g_i = o2 + lax.dot_general(p32, rw_ref[...], dnm, preferred_element_type=jnp.float32)
    g_n = lax.dot_general(p21, bm_ref[...], dnm, preferred_element_type=jnp.float32)
    c = (lax.dot_general(tib_ref[...], p21, dnt, preferred_element_type=jnp.float32)
         + lax.dot_general(trb_ref[...], p32, dnt, preferred_element_type=jnp.float32)
         + ob_ref[...])

    x = (lax.dot_general(s, g_s, dnt, preferred_element_type=jnp.float32)
         + lax.dot_general(i, g_i, dnt, preferred_element_type=jnp.float32)
         + lax.dot_general(r, o3, dnt, preferred_element_type=jnp.float32)
         + lax.dot_general(nb, g_n, dnt, preferred_element_type=jnp.float32)
         + c)
    x = x - jnp.max(x, axis=-1, keepdims=True)
    e = jnp.exp(x)
    out_ref[...] = e / jnp.sum(e, axis=-1, keepdims=True)


@functools.cache
def _make_sc_scatter(n, d, e, nc, ns, c):
    nw = nc * ns
    epw = e // nw
    nchunk = epw // c
    npc = 25
    nstage = nchunk // npc
    rpt = (n // ns) // 8 * 8
    tail = n - rpt * ns
    mesh = plsc.VectorSubcoreMesh(core_axis_name="c", subcore_axis_name="s")

    @functools.partial(
        pl.kernel,
        out_type=jax.ShapeDtypeStruct((nc, n, d), jnp.float32),
        mesh=mesh,
        scratch_types=[
            pltpu.VMEM((npc, c), jnp.int32),
            pltpu.VMEM((npc, c), jnp.int32),
            pltpu.VMEM((npc, c), jnp.float32),
            pltpu.VMEM((2, d), jnp.float32),
            pltpu.VMEM((c, d), jnp.float32),
            pltpu.VMEM((c, d), jnp.float32),
            pltpu.VMEM_SHARED((n, d), jnp.float32),
            pltpu.SemaphoreType.DMA,
            pltpu.SemaphoreType.DMA,
        ],
    )
    def sc_scatter(zi_hbm, coef_hbm, src_hbm, dst_hbm, w_hbm, out_hbm,
                   src_v, dst_v, w_v, coef_v, rows0_v, rows1_v, acc,
                   gs0, gs1):
        ci = lax.axis_index("c")
        si = lax.axis_index("s")
        wid = si * nc + ci
        r0 = pl.multiple_of(si * rpt, 8)
        pltpu.sync_copy(coef_hbm, coef_v)

        def zrow(rr, carry):
            for k in range(d // 16):
                rows0_v[rr, pl.ds(k * 16, 16)] = jnp.zeros((16,), jnp.float32)
            return carry

        lax.fori_loop(0, c, zrow, 0)
        for t in range(rpt // c):
            pltpu.sync_copy(rows0_v, acc.at[pl.ds(r0 + t * c, c)])
        rem = rpt % c
        if rem:
            pltpu.sync_copy(rows0_v.at[pl.ds(0, rem)],
                            acc.at[pl.ds(r0 + (rpt // c) * c, rem)])
        if tail:
            @pl.when(si == ns - 1)
            def _():
                pltpu.sync_copy(rows0_v.at[pl.ds(0, tail)],
                                acc.at[pl.ds(rpt * ns, tail)])
        plsc.subcore_barrier()

        scv = [coef_v[0, pl.ds(k * 16, 16)] for k in range(d // 16)]
        shv = [coef_v[1, pl.ds(k * 16, 16)] for k in range(d // 16)]

        rows = (rows0_v, rows1_v)
        gsem = (gs0, gs1)

        def stage(st, carry):
            pltpu.sync_copy(src_hbm.at[wid, st], src_v)
            pltpu.sync_copy(dst_hbm.at[wid, st], dst_v)
            pltpu.sync_copy(w_hbm.at[wid, st], w_v)
            pltpu.async_copy(zi_hbm.at[src_v.at[0]], rows0_v, gs0)

            def body(j, bi):
                cur = rows[bi]
                ni = 1 - bi
                pltpu.make_async_copy(zi_hbm.at[src_v.at[j]], cur, gsem[bi]).wait()

                @pl.when(j < npc - 1)
                def _():
                    pltpu.async_copy(zi_hbm.at[src_v.at[j + 1]], rows[ni], gsem[ni])

                def group(g, c3):
                    go = pl.multiple_of(g * 16, 16)
                    wv = w_v[j, pl.ds(go, 16)]
                    for e in range(16):
                        w = wv[e]
                        ei = go + e
                        for k in range(d // 16):
                            sl = pl.ds(k * 16, 16)
                            z = cur[ei, sl]
                            cur[ei, sl] = jnp.maximum(
                                z * scv[k] + shv[k], 0.0) * w
                    return c3

                lax.fori_loop(0, c // 16, group, 0)
                pltpu.sync_copy(cur, acc.at[dst_v.at[j]], add=True)

            def chunk(j, c2):
                for b in range(2):
                    @pl.when(j % 2 == b)
                    def _(b=b):
                        body(j, b)

                return c2

            lax.fori_loop(0, npc, chunk, 0)
            return carry

        lax.fori_loop(0, nstage, stage, 0)
        plsc.subcore_barrier()
        pltpu.sync_copy(acc.at[pl.ds(r0, rpt)], out_hbm.at[ci, pl.ds(r0, rpt)])
        if tail:
            @pl.when(si == ns - 1)
            def _():
                pltpu.sync_copy(acc.at[pl.ds(rpt * ns, tail)],
                                out_hbm.at[ci, pl.ds(rpt * ns, tail)])

    return sc_scatter


def _sc_partials(zi, coefs, src3, dst3, w3, nc, ns, c):
    n, d = zi.shape
    e = src3.size
    return _make_sc_scatter(n, d, e, nc, ns, c)(zi, coefs, src3, dst3, w3)


def kernel(feature, edge_index, edge_weight, W_s, b_s, W_i, b_i, W_r, b_r,
           bn_gamma, bn_beta, toI_W, toI_b, toR_W, toR_b, out_W, out_b):
    n, d = feature.shape
    e = edge_weight.shape[0]
    blk = 1000 if n % 1000 == 0 else n
    nblk = n // blk
    nf = float(n)

    bs = b_s.reshape(1, d)
    bi = b_i.reshape(1, d)
    br = b_r.reshape(1, d)
    gam = bn_gamma.reshape(1, d)
    bet = bn_beta.reshape(1, d)

    row = lambda i: pl.BlockSpec((blk, d), lambda b: (b, 0))
    full = lambda s: pl.BlockSpec(s, lambda b: (0,) * len(s))

    z_s, z_i, z_r, stats = pl.pallas_call(
        _k1_body,
        grid=(nblk,),
        in_specs=[row(0)] + [full((d, d))] * 3 + [full((1, d))] * 3,
        out_specs=[row(0), row(0), row(0), full((8, d))],
        out_shape=[jax.ShapeDtypeStruct((n, d), jnp.float32)] * 3
        + [jax.ShapeDtypeStruct((8, d), jnp.float32)],
    )(feature, W_s, W_i, W_r, bs, bi, br)

    mean_i = stats[2] * (1.0 / nf)
    var_i = stats[3] * (1.0 / nf) - mean_i * mean_i
    sc_i = bn_gamma * lax.rsqrt(var_i + _EPS)
    sh_i = bn_beta - mean_i * sc_i
    coefs = jnp.stack([sc_i, sh_i])

    info = plsc.get_sparse_core_info()
    nc, ns = info.num_cores, info.num_subcores
    c = 80
    npc = 25
    src3 = edge_index[0].reshape(nc * ns, -1, npc, c)
    dst3 = edge_index[1].reshape(nc * ns, -1, npc, c)
    w3 = edge_weight.reshape(nc * ns, -1, npc, c)
    partials = _sc_partials(z_i, coefs, src3, dst3, w3, nc, ns, c)

    a_m = toI_W[:, :d]
    b_m = toI_W[:, d:]
    o1 = out_W[:, :d]
    o2 = out_W[:, d:2 * d]
    o3 = out_W[:, 2 * d:]

    out = pl.pallas_call(
        functools.partial(_k3_body, nf),
        grid=(nblk,),
        in_specs=[row(0)] * 5 + [full((8, d)), full((1, d)), full((1, d))]
        + [full((d, d))] * 3 + [full((3, d))] * 3
        + [full((1, d)), full((1, d)), full((1, 3))],
        out_specs=pl.BlockSpec((blk, 3), lambda b: (b, 0)),
        out_shape=jax.ShapeDtypeStruct((n, 3), jnp.float32),
    )(z_s, z_r, z_i, partials[0], partials[1], stats, gam, bet,
      a_m, b_m, toR_W, o1, o2, o3,
      toI_b.reshape(1, d), toR_b.reshape(1, d), out_b.reshape(1, 3))
    return out


# speedup vs baseline: 1.1726x; 1.0002x over previous
import functools

import jax
import jax.numpy as jnp
from jax import lax
from jax.experimental import pallas as pl
from jax.experimental.pallas import tpu as pltpu
from jax.experimental.pallas import tpu_sc as plsc

_EPS = 1e-5


def _k1_body(f_ref, ws_ref, wi_ref, wr_ref, bs_ref, bi_ref, br_ref,
             zs_ref, zi_ref, zr_ref, stats_ref):
    @pl.when(pl.program_id(0) == 0)
    def _():
        stats_ref[...] = jnp.zeros_like(stats_ref)

    f = f_ref[...]
    dn = (((1,), (1,)), ((), ()))
    zs = lax.dot_general(f, ws_ref[...], dn, preferred_element_type=jnp.float32) + f + bs_ref[...]
    zi = lax.dot_general(f, wi_ref[...], dn, preferred_element_type=jnp.float32) + f + bi_ref[...]
    zr = lax.dot_general(f, wr_ref[...], dn, preferred_element_type=jnp.float32) + f + br_ref[...]
    zs_ref[...] = zs
    zi_ref[...] = zi
    zr_ref[...] = zr
    upd = jnp.concatenate(
        [jnp.sum(zs, 0, keepdims=True), jnp.sum(zs * zs, 0, keepdims=True),
         jnp.sum(zi, 0, keepdims=True), jnp.sum(zi * zi, 0, keepdims=True),
         jnp.sum(zr, 0, keepdims=True), jnp.sum(zr * zr, 0, keepdims=True),
         jnp.zeros((2, zs.shape[1]), jnp.float32)], axis=0)
    stats_ref[...] += upd


def _bn_coefs(stats, row, gamma, beta, n):
    mean = stats[row:row + 1, :] * (1.0 / n)
    var = stats[row + 1:row + 2, :] * (1.0 / n) - mean * mean
    sc = gamma * lax.rsqrt(var + _EPS)
    sh = beta - mean * sc
    return sc, sh


def _k3_body(n, zs_ref, zr_ref, zi_ref, n0_ref, n1_ref, stats_ref, g_ref, b_ref,
             a_ref, bm_ref, rw_ref, o1_ref, o2_ref, o3_ref,
             tib_ref, trb_ref, ob_ref, out_ref):
    st = stats_ref[...]
    g = g_ref[...]
    be = b_ref[...]
    sc_s, sh_s = _bn_coefs(st, 0, g, be, n)
    sc_i, sh_i = _bn_coefs(st, 2, g, be, n)
    sc_r, sh_r = _bn_coefs(st, 4, g, be, n)
    s = jnp.maximum(zs_ref[...] * sc_s + sh_s, 0.0)
    r = jnp.maximum(zr_ref[...] * sc_r + sh_r, 0.0)
    i = jnp.maximum(zi_ref[...] * sc_i + sh_i, 0.0)
    nb = n0_ref[...] + n1_ref[...]

    o1 = o1_ref[...]
    o2 = o2_ref[...]
    o3 = o3_ref[...]
    p21 = o2 - o1
    p32 = o3 - o2
    dnm = (((1,), (0,)), ((), ()))
    dnt = (((1,), (1,)), ((), ()))
    g_s = o1 + lax.dot_general(p21, a_ref[...], dnm, preferred_element_type=jnp.float32)
    g_i = o2 + lax.dot_general(p32, rw_ref[...], dnm, preferred_element_type=jnp.float32)
    g_n = lax.dot_general(p21, bm_ref[...], dnm, preferred_element_type=jnp.float32)
    c = (lax.dot_general(tib_ref[...], p21, dnt, preferred_element_type=jnp.float32)
         + lax.dot_general(trb_ref[...], p32, dnt, preferred_element_type=jnp.float32)
         + ob_ref[...])

    x = (lax.dot_general(s, g_s, dnt, preferred_element_type=jnp.float32)
         + lax.dot_general(i, g_i, dnt, preferred_element_type=jnp.float32)
         + lax.dot_general(r, o3, dnt, preferred_element_type=jnp.float32)
         + lax.dot_general(nb, g_n, dnt, preferred_element_type=jnp.float32)
         + c)
    x = x - jnp.max(x, axis=-1, keepdims=True)
    e = jnp.exp(x)
    out_ref[...] = e / jnp.sum(e, axis=-1, keepdims=True)


@functools.cache
def _make_sc_scatter(n, d, e, nc, ns, c):
    nw = nc * ns
    epw = e // nw
    nchunk = epw // c
    npc = 25
    nstage = nchunk // npc
    rpt = (n // ns) // 8 * 8
    tail = n - rpt * ns
    mesh = plsc.VectorSubcoreMesh(core_axis_name="c", subcore_axis_name="s")

    @functools.partial(
        pl.kernel,
        out_type=jax.ShapeDtypeStruct((nc, n, d), jnp.float32),
        mesh=mesh,
        scratch_types=[
            pltpu.VMEM((npc, c), jnp.int32),
            pltpu.VMEM((npc, c), jnp.int32),
            pltpu.VMEM((npc, c), jnp.float32),
            pltpu.VMEM((2, d), jnp.float32),
            pltpu.VMEM((c, d), jnp.float32),
            pltpu.VMEM((c, d), jnp.float32),
            pltpu.VMEM((c, d), jnp.float32),
            pltpu.VMEM_SHARED((n, d), jnp.float32),
            pltpu.SemaphoreType.DMA,
            pltpu.SemaphoreType.DMA,
            pltpu.SemaphoreType.DMA,
        ],
    )
    def sc_scatter(zi_hbm, coef_hbm, src_hbm, dst_hbm, w_hbm, out_hbm,
                   src_v, dst_v, w_v, coef_v, rows0_v, rows1_v, rows2_v, acc,
                   gs0, gs1, gs2):
        ci = lax.axis_index("c")
        si = lax.axis_index("s")
        wid = si * nc + ci
        r0 = pl.multiple_of(si * rpt, 8)
        pltpu.sync_copy(coef_hbm, coef_v)

        def zrow(rr, carry):
            for k in range(d // 16):
                rows0_v[rr, pl.ds(k * 16, 16)] = jnp.zeros((16,), jnp.float32)
            return carry

        lax.fori_loop(0, c, zrow, 0)
        for t in range(rpt // c):
            pltpu.sync_copy(rows0_v, acc.at[pl.ds(r0 + t * c, c)])
        rem = rpt % c
        if rem:
            pltpu.sync_copy(rows0_v.at[pl.ds(0, rem)],
                            acc.at[pl.ds(r0 + (rpt // c) * c, rem)])
        if tail:
            @pl.when(si == ns - 1)
            def _():
                pltpu.sync_copy(rows0_v.at[pl.ds(0, tail)],
                                acc.at[pl.ds(rpt * ns, tail)])
        plsc.subcore_barrier()

        scv = [coef_v[0, pl.ds(k * 16, 16)] for k in range(d // 16)]
        shv = [coef_v[1, pl.ds(k * 16, 16)] for k in range(d // 16)]

        rows = (rows0_v, rows1_v, rows2_v)
        gsem = (gs0, gs1, gs2)

        def stage(st, carry):
            pltpu.sync_copy(src_hbm.at[wid, st], src_v)
            pltpu.sync_copy(dst_hbm.at[wid, st], dst_v)
            pltpu.sync_copy(w_hbm.at[wid, st], w_v)
            pltpu.async_copy(zi_hbm.at[src_v.at[0]], rows0_v, gs0)
            pltpu.async_copy(zi_hbm.at[src_v.at[1]], rows1_v, gs1)

            def body(j, bi):
                cur = rows[bi]
                ni = (bi + 2) % 3
                pltpu.make_async_copy(zi_hbm.at[src_v.at[j]], cur, gsem[bi]).wait()

                @pl.when(j < npc - 2)
                def _():
                    pltpu.async_copy(zi_hbm.at[src_v.at[j + 2]], rows[ni], gsem[ni])

                def group(g, c3):
                    go = pl.multiple_of(g * 16, 16)
                    wv = w_v[j, pl.ds(go, 16)]
                    for e in range(16):
                        w = wv[e]
                        ei = go + e
                        for k in range(d // 16):
                            sl = pl.ds(k * 16, 16)
                            z = cur[ei, sl]
                            cur[ei, sl] = jnp.maximum(
                                z * scv[k] + shv[k], 0.0) * w
                    return c3

                lax.fori_loop(0, c // 16, group, 0)
                pltpu.sync_copy(cur, acc.at[dst_v.at[j]], add=True)

            def chunk(j, c2):
                for b in range(3):
                    @pl.when(j % 3 == b)
                    def _(b=b):
                        body(j, b)

                return c2

            lax.fori_loop(0, npc, chunk, 0)
            return carry

        lax.fori_loop(0, nstage, stage, 0)
        plsc.subcore_barrier()
        pltpu.sync_copy(acc.at[pl.ds(r0, rpt)], out_hbm.at[ci, pl.ds(r0, rpt)])
        if tail:
            @pl.when(si == ns - 1)
            def _():
                pltpu.sync_copy(acc.at[pl.ds(rpt * ns, tail)],
                                out_hbm.at[ci, pl.ds(rpt * ns, tail)])

    return sc_scatter


def _sc_partials(zi, coefs, src3, dst3, w3, nc, ns, c):
    n, d = zi.shape
    e = src3.size
    return _make_sc_scatter(n, d, e, nc, ns, c)(zi, coefs, src3, dst3, w3)


def kernel(feature, edge_index, edge_weight, W_s, b_s, W_i, b_i, W_r, b_r,
           bn_gamma, bn_beta, toI_W, toI_b, toR_W, toR_b, out_W, out_b):
    n, d = feature.shape
    e = edge_weight.shape[0]
    blk = 1000 if n % 1000 == 0 else n
    nblk = n // blk
    nf = float(n)

    bs = b_s.reshape(1, d)
    bi = b_i.reshape(1, d)
    br = b_r.reshape(1, d)
    gam = bn_gamma.reshape(1, d)
    bet = bn_beta.reshape(1, d)

    row = lambda i: pl.BlockSpec((blk, d), lambda b: (b, 0))
    full = lambda s: pl.BlockSpec(s, lambda b: (0,) * len(s))

    z_s, z_i, z_r, stats = pl.pallas_call(
        _k1_body,
        grid=(nblk,),
        in_specs=[row(0)] + [full((d, d))] * 3 + [full((1, d))] * 3,
        out_specs=[row(0), row(0), row(0), full((8, d))],
        out_shape=[jax.ShapeDtypeStruct((n, d), jnp.float32)] * 3
        + [jax.ShapeDtypeStruct((8, d), jnp.float32)],
    )(feature, W_s, W_i, W_r, bs, bi, br)

    mean_i = stats[2] * (1.0 / nf)
    var_i = stats[3] * (1.0 / nf) - mean_i * mean_i
    sc_i = bn_gamma * lax.rsqrt(var_i + _EPS)
    sh_i = bn_beta - mean_i * sc_i
    coefs = jnp.stack([sc_i, sh_i])

    info = plsc.get_sparse_core_info()
    nc, ns = info.num_cores, info.num_subcores
    c = 80
    npc = 25
    src3 = edge_index[0].reshape(nc * ns, -1, npc, c)
    dst3 = edge_index[1].reshape(nc * ns, -1, npc, c)
    w3 = edge_weight.reshape(nc * ns, -1, npc, c)
    partials = _sc_partials(z_i, coefs, src3, dst3, w3, nc, ns, c)

    a_m = toI_W[:, :d]
    b_m = toI_W[:, d:]
    o1 = out_W[:, :d]
    o2 = out_W[:, d:2 * d]
    o3 = out_W[:, 2 * d:]

    out = pl.pallas_call(
        functools.partial(_k3_body, nf),
        grid=(nblk,),
        in_specs=[row(0)] * 5 + [full((8, d)), full((1, d)), full((1, d))]
        + [full((d, d))] * 3 + [full((3, d))] * 3
        + [full((1, d)), full((1, d)), full((1, 3))],
        out_specs=pl.BlockSpec((blk, 3), lambda b: (b, 0)),
        out_shape=jax.ShapeDtypeStruct((n, 3), jnp.float32),
    )(z_s, z_r, z_i, partials[0], partials[1], stats, gam, bet,
      a_m, b_m, toR_W, o1, o2, o3,
      toI_b.reshape(1, d), toR_b.reshape(1, d), out_b.reshape(1, 3))
    return out


# speedup vs baseline: 1.1803x; 1.0065x over previous
import functools

import jax
import jax.numpy as jnp
from jax import lax
from jax.experimental import pallas as pl
from jax.experimental.pallas import tpu as pltpu
from jax.experimental.pallas import tpu_sc as plsc

_EPS = 1e-5


def _k1_body(nb, nf, f_ref, ws_ref, wi_ref, wr_ref, bs_ref, bi_ref, br_ref,
             g_ref, be_ref, zs_ref, zi_ref, zr_ref, stats_ref):
    @pl.when(pl.program_id(0) == 0)
    def _():
        stats_ref[...] = jnp.zeros_like(stats_ref)

    f = f_ref[...]
    dn = (((1,), (1,)), ((), ()))
    zs = lax.dot_general(f, ws_ref[...], dn, preferred_element_type=jnp.float32) + f + bs_ref[...]
    zi = lax.dot_general(f, wi_ref[...], dn, preferred_element_type=jnp.float32) + f + bi_ref[...]
    zr = lax.dot_general(f, wr_ref[...], dn, preferred_element_type=jnp.float32) + f + br_ref[...]
    zs_ref[...] = zs
    zi_ref[...] = zi
    zr_ref[...] = zr
    upd = jnp.concatenate(
        [jnp.sum(zs, 0, keepdims=True), jnp.sum(zs * zs, 0, keepdims=True),
         jnp.sum(zi, 0, keepdims=True), jnp.sum(zi * zi, 0, keepdims=True),
         jnp.sum(zr, 0, keepdims=True), jnp.sum(zr * zr, 0, keepdims=True),
         jnp.zeros((2, zs.shape[1]), jnp.float32)], axis=0)
    stats_ref[...] += upd

    @pl.when(pl.program_id(0) == nb - 1)
    def _():
        sc_i, sh_i = _bn_coefs(stats_ref[...], 2, g_ref[...], be_ref[...], nf)
        stats_ref[6:7, :] = sc_i
        stats_ref[7:8, :] = sh_i


def _bn_coefs(stats, row, gamma, beta, n):
    mean = stats[row:row + 1, :] * (1.0 / n)
    var = stats[row + 1:row + 2, :] * (1.0 / n) - mean * mean
    sc = gamma * lax.rsqrt(var + _EPS)
    sh = beta - mean * sc
    return sc, sh


def _k3_body(n, zs_ref, zr_ref, zi_ref, n0_ref, n1_ref, stats_ref, g_ref, b_ref,
             a_ref, bm_ref, rw_ref, o1_ref, o2_ref, o3_ref,
             tib_ref, trb_ref, ob_ref, out_ref):
    st = stats_ref[...]
    g = g_ref[...]
    be = b_ref[...]
    sc_s, sh_s = _bn_coefs(st, 0, g, be, n)
    sc_i, sh_i = _bn_coefs(st, 2, g, be, n)
    sc_r, sh_r = _bn_coefs(st, 4, g, be, n)
    s = jnp.maximum(zs_ref[...] * sc_s + sh_s, 0.0)
    r = jnp.maximum(zr_ref[...] * sc_r + sh_r, 0.0)
    i = jnp.maximum(zi_ref[...] * sc_i + sh_i, 0.0)
    nb = n0_ref[...] + n1_ref[...]

    o1 = o1_ref[...]
    o2 = o2_ref[...]
    o3 = o3_ref[...]
    p21 = o2 - o1
    p32 = o3 - o2
    dnm = (((1,), (0,)), ((), ()))
    dnt = (((1,), (1,)), ((), ()))
    g_s = o1 + lax.dot_general(p21, a_ref[...], dnm, preferred_element_type=jnp.float32)
    g_i = o2 + lax.dot_general(p32, rw_ref[...], dnm, preferred_element_type=jnp.float32)
    g_n = lax.dot_general(p21, bm_ref[...], dnm, preferred_element_type=jnp.float32)
    c = (lax.dot_general(tib_ref[...], p21, dnt, preferred_element_type=jnp.float32)
         + lax.dot_general(trb_ref[...], p32, dnt, preferred_element_type=jnp.float32)
         + ob_ref[...])

    x = (lax.dot_general(s, g_s, dnt, preferred_element_type=jnp.float32)
         + lax.dot_general(i, g_i, dnt, preferred_element_type=jnp.float32)
         + lax.dot_general(r, o3, dnt, preferred_element_type=jnp.float32)
         + lax.dot_general(nb, g_n, dnt, preferred_element_type=jnp.float32)
         + c)
    x = x - jnp.max(x, axis=-1, keepdims=True)
    e = jnp.exp(x)
    out_ref[...] = e / jnp.sum(e, axis=-1, keepdims=True)


@functools.cache
def _make_sc_scatter(n, d, e, nc, ns, c):
    nw = nc * ns
    epw = e // nw
    nchunk = epw // c
    npc = 25
    nstage = nchunk // npc
    rpt = (n // ns) // 8 * 8
    tail = n - rpt * ns
    mesh = plsc.VectorSubcoreMesh(core_axis_name="c", subcore_axis_name="s")

    @functools.partial(
        pl.kernel,
        out_type=jax.ShapeDtypeStruct((nc, n, d), jnp.float32),
        mesh=mesh,
        scratch_types=[
            pltpu.VMEM((npc, c), jnp.int32),
            pltpu.VMEM((npc, c), jnp.int32),
            pltpu.VMEM((npc, c), jnp.float32),
            pltpu.VMEM((8, d), jnp.float32),
            pltpu.VMEM((c, d), jnp.float32),
            pltpu.VMEM((c, d), jnp.float32),
            pltpu.VMEM((c, d), jnp.float32),
            pltpu.VMEM_SHARED((n, d), jnp.float32),
            pltpu.SemaphoreType.DMA,
            pltpu.SemaphoreType.DMA,
            pltpu.SemaphoreType.DMA,
        ],
    )
    def sc_scatter(zi_hbm, coef_hbm, src_hbm, dst_hbm, w_hbm, out_hbm,
                   src_v, dst_v, w_v, coef_v, rows0_v, rows1_v, rows2_v, acc,
                   gs0, gs1, gs2):
        ci = lax.axis_index("c")
        si = lax.axis_index("s")
        wid = si * nc + ci
        r0 = pl.multiple_of(si * rpt, 8)
        pltpu.sync_copy(coef_hbm, coef_v)

        def zrow(rr, carry):
            for k in range(d // 16):
                rows0_v[rr, pl.ds(k * 16, 16)] = jnp.zeros((16,), jnp.float32)
            return carry

        lax.fori_loop(0, c, zrow, 0)
        for t in range(rpt // c):
            pltpu.sync_copy(rows0_v, acc.at[pl.ds(r0 + t * c, c)])
        rem = rpt % c
        if rem:
            pltpu.sync_copy(rows0_v.at[pl.ds(0, rem)],
                            acc.at[pl.ds(r0 + (rpt // c) * c, rem)])
        if tail:
            @pl.when(si == ns - 1)
            def _():
                pltpu.sync_copy(rows0_v.at[pl.ds(0, tail)],
                                acc.at[pl.ds(rpt * ns, tail)])
        plsc.subcore_barrier()

        scv = [coef_v[6, pl.ds(k * 16, 16)] for k in range(d // 16)]
        shv = [coef_v[7, pl.ds(k * 16, 16)] for k in range(d // 16)]

        rows = (rows0_v, rows1_v, rows2_v)
        gsem = (gs0, gs1, gs2)

        def stage(st, carry):
            pltpu.sync_copy(src_hbm.at[wid, st], src_v)
            pltpu.sync_copy(dst_hbm.at[wid, st], dst_v)
            pltpu.sync_copy(w_hbm.at[wid, st], w_v)
            pltpu.async_copy(zi_hbm.at[src_v.at[0]], rows0_v, gs0)
            pltpu.async_copy(zi_hbm.at[src_v.at[1]], rows1_v, gs1)

            def body(j, bi):
                cur = rows[bi]
                ni = (bi + 2) % 3
                pltpu.make_async_copy(zi_hbm.at[src_v.at[j]], cur, gsem[bi]).wait()

                @pl.when(j < npc - 2)
                def _():
                    pltpu.async_copy(zi_hbm.at[src_v.at[j + 2]], rows[ni], gsem[ni])

                def group(g, c3):
                    go = pl.multiple_of(g * 16, 16)
                    wv = w_v[j, pl.ds(go, 16)]
                    for e in range(16):
                        w = wv[e]
                        ei = go + e
                        for k in range(d // 16):
                            sl = pl.ds(k * 16, 16)
                            z = cur[ei, sl]
                            cur[ei, sl] = jnp.maximum(
                                z * scv[k] + shv[k], 0.0) * w
                    return c3

                lax.fori_loop(0, c // 16, group, 0)
                pltpu.sync_copy(cur, acc.at[dst_v.at[j]], add=True)

            def chunk(j, c2):
                for b in range(3):
                    @pl.when(j % 3 == b)
                    def _(b=b):
                        body(j, b)

                return c2

            lax.fori_loop(0, npc, chunk, 0)
            return carry

        lax.fori_loop(0, nstage, stage, 0)
        plsc.subcore_barrier()
        pltpu.sync_copy(acc.at[pl.ds(r0, rpt)], out_hbm.at[ci, pl.ds(r0, rpt)])
        if tail:
            @pl.when(si == ns - 1)
            def _():
                pltpu.sync_copy(acc.at[pl.ds(rpt * ns, tail)],
                                out_hbm.at[ci, pl.ds(rpt * ns, tail)])

    return sc_scatter


def _sc_partials(zi, coefs, src3, dst3, w3, nc, ns, c):
    n, d = zi.shape
    e = src3.size
    return _make_sc_scatter(n, d, e, nc, ns, c)(zi, coefs, src3, dst3, w3)


def kernel(feature, edge_index, edge_weight, W_s, b_s, W_i, b_i, W_r, b_r,
           bn_gamma, bn_beta, toI_W, toI_b, toR_W, toR_b, out_W, out_b):
    n, d = feature.shape
    e = edge_weight.shape[0]
    blk = 1000 if n % 1000 == 0 else n
    nblk = n // blk
    nf = float(n)

    bs = b_s.reshape(1, d)
    bi = b_i.reshape(1, d)
    br = b_r.reshape(1, d)
    gam = bn_gamma.reshape(1, d)
    bet = bn_beta.reshape(1, d)

    row = lambda i: pl.BlockSpec((blk, d), lambda b: (b, 0))
    full = lambda s: pl.BlockSpec(s, lambda b: (0,) * len(s))

    z_s, z_i, z_r, stats = pl.pallas_call(
        functools.partial(_k1_body, nblk, nf),
        grid=(nblk,),
        in_specs=[row(0)] + [full((d, d))] * 3 + [full((1, d))] * 5,
        out_specs=[row(0), row(0), row(0), full((8, d))],
        out_shape=[jax.ShapeDtypeStruct((n, d), jnp.float32)] * 3
        + [jax.ShapeDtypeStruct((8, d), jnp.float32)],
    )(feature, W_s, W_i, W_r, bs, bi, br, gam, bet)

    info = plsc.get_sparse_core_info()
    nc, ns = info.num_cores, info.num_subcores
    c = 80
    npc = 25
    src3 = edge_index[0].reshape(nc * ns, -1, npc, c)
    dst3 = edge_index[1].reshape(nc * ns, -1, npc, c)
    w3 = edge_weight.reshape(nc * ns, -1, npc, c)
    partials = _sc_partials(z_i, stats, src3, dst3, w3, nc, ns, c)

    a_m = toI_W[:, :d]
    b_m = toI_W[:, d:]
    o1 = out_W[:, :d]
    o2 = out_W[:, d:2 * d]
    o3 = out_W[:, 2 * d:]

    out = pl.pallas_call(
        functools.partial(_k3_body, nf),
        grid=(nblk,),
        in_specs=[row(0)] * 5 + [full((8, d)), full((1, d)), full((1, d))]
        + [full((d, d))] * 3 + [full((3, d))] * 3
        + [full((1, d)), full((1, d)), full((1, 3))],
        out_specs=pl.BlockSpec((blk, 3), lambda b: (b, 0)),
        out_shape=jax.ShapeDtypeStruct((n, 3), jnp.float32),
    )(z_s, z_r, z_i, partials[0], partials[1], stats, gam, bet,
      a_m, b_m, toR_W, o1, o2, o3,
      toI_b.reshape(1, d), toR_b.reshape(1, d), out_b.reshape(1, 3))
    return out
